# Initial kernel scaffold; baseline (speedup 1.0000x reference)
import jax
import jax.numpy as jnp
from jax.experimental import pallas as pl


def kernel(seg0, seg1, len0, len1):
    raise NotImplementedError("write your pallas kernel here")


# speedup vs baseline: 5.4198x; 5.4198x over previous
import jax
import jax.numpy as jnp
from jax import lax
from jax.experimental import pallas as pl
from jax.experimental.pallas import tpu as pltpu
from jax.experimental.pallas import tpu_sc as plsc

SEQ_LEN = 512
START = 0
END = 2
PAD = 1
B, L = 4096, 384
BUDGET = SEQ_LEN - 4
FAIR0 = (BUDGET + 1) // 2
FAIR1 = BUDGET // 2

NC = 2
NS = 16
LANES = 16
NW = NC * NS
ROWS_PER_W = B // NW
ROW_BLK = 16
N_BLKS = ROWS_PER_W // ROW_BLK
N_CHUNKS = SEQ_LEN // LANES


def _body(seg0_hbm, seg1_hbm, len0_hbm, len1_hbm, out_hbm,
          s0_v, s1_v, l0_v, l1_v, outblk_v):
    wid = lax.axis_index("s") * NC + lax.axis_index("c")
    base = wid * ROWS_PER_W

    pltpu.sync_copy(seg0_hbm.at[pl.ds(base * L, ROWS_PER_W * L)], s0_v)
    pltpu.sync_copy(seg1_hbm.at[pl.ds(base * L, ROWS_PER_W * L)], s1_v)
    pltpu.sync_copy(len0_hbm.at[pl.ds(base, ROWS_PER_W)], l0_v)
    pltpu.sync_copy(len1_hbm.at[pl.ds(base, ROWS_PER_W)], l1_v)

    iota = lax.iota(jnp.int32, LANES)
    pad_v = jnp.full((LANES,), PAD, jnp.int32)
    end_v = jnp.full((LANES,), END, jnp.int32)
    start_v = jnp.full((LANES,), START, jnp.int32)

    def do_blk(blk, _):
        lv0 = l0_v[pl.ds(blk * ROW_BLK, ROW_BLK)]
        lv1 = l1_v[pl.ds(blk * ROW_BLK, ROW_BLK)]
        k0vec = jnp.minimum(lv0, jnp.maximum(FAIR0, BUDGET - lv1))
        k1vec = jnp.minimum(lv1, jnp.maximum(FAIR1, BUDGET - lv0))
        tvec = k0vec + 3 + k1vec

        for r16 in range(ROW_BLK):
            k0 = k0vec[r16]
            t = tvec[r16]
            k03 = t - k1vec[r16]
            n_work = (t + LANES) >> 4
            r = blk * ROW_BLK + r16

            k0v = jnp.full((LANES,), k0, jnp.int32)
            k03v = jnp.full((LANES,), k03, jnp.int32)
            tv = jnp.full((LANES,), t, jnp.int32)
            rowbase0 = jnp.full((LANES,), r * L, jnp.int32)
            outbase = jnp.full((LANES,), r16 * SEQ_LEN, jnp.int32)

            def work_chunk(c, _, k0v=k0v, k03v=k03v, tv=tv,
                           rowbase0=rowbase0, outbase=outbase):
                j = iota + c * LANES
                idx0 = jnp.clip(j - 1, 0, L - 1)
                g0 = plsc.load_gather(s0_v, [rowbase0 + idx0])
                idx1 = jnp.clip(j - k03v, 0, L - 1)
                g1 = plsc.load_gather(s1_v, [rowbase0 + idx1])
                val = jnp.where(
                    j <= k0v, g0,
                    jnp.where(j < k03v, end_v,
                              jnp.where(j < tv, g1,
                                        jnp.where(j == tv, end_v, pad_v))))
                val = jnp.where(j == 0, start_v, val)
                plsc.store_scatter(outblk_v, [outbase + j], val)
                return 0

            def pad_chunk(c, _, outbase=outbase):
                j = iota + c * LANES
                plsc.store_scatter(outblk_v, [outbase + j], pad_v)
                return 0

            lax.fori_loop(0, n_work, work_chunk, 0)
            lax.fori_loop(n_work, N_CHUNKS, pad_chunk, 0)

        pltpu.sync_copy(
            outblk_v,
            out_hbm.at[pl.ds((base + blk * ROW_BLK) * SEQ_LEN, ROW_BLK * SEQ_LEN)])
        return 0

    lax.fori_loop(0, N_BLKS, do_blk, 0)


@jax.jit
def kernel(seg0, seg1, len0, len1):
    mesh = plsc.VectorSubcoreMesh(
        core_axis_name="c", subcore_axis_name="s", num_cores=NC, num_subcores=NS)
    f = pl.kernel(
        _body,
        out_type=jax.ShapeDtypeStruct((B * SEQ_LEN,), jnp.int32),
        mesh=mesh,
        compiler_params=pltpu.CompilerParams(needs_layout_passes=False),
        scratch_types=[
            pltpu.VMEM((ROWS_PER_W * L,), jnp.int32),
            pltpu.VMEM((ROWS_PER_W * L,), jnp.int32),
            pltpu.VMEM((ROWS_PER_W,), jnp.int32),
            pltpu.VMEM((ROWS_PER_W,), jnp.int32),
            pltpu.VMEM((ROW_BLK * SEQ_LEN,), jnp.int32),
        ],
    )
    out = f(seg0.reshape(B * L), seg1.reshape(B * L), len0, len1)
    return out.reshape(B, SEQ_LEN)
